# Initial kernel scaffold; baseline (speedup 1.0000x reference)
import jax
import jax.numpy as jnp
from jax.experimental import pallas as pl


def kernel(images, uncertainty_maps):
    raise NotImplementedError("write your pallas kernel here")


# speedup vs baseline: 1.4082x; 1.4082x over previous
import functools
import jax
import jax.numpy as jnp
from jax import lax
from jax.experimental import pallas as pl
from jax.experimental.pallas import tpu as pltpu
from jax.experimental.pallas import tpu_sc as plsc

H = 512
W = 512
EFF = 64
HALF = 32
MARGIN = 16
NPATCH = 3
CHANS = 3
BATCH = 16
NSEG = 16
SEGW = 32
CHUNK = 96
SROWS = 104
PSZ = EFF * EFF
NEG = float(-3.4028235e38)
BIGI = 1 << 30


def _worker(b, images, unc, patches_out, coords_out, staging, rowseg, pbuf, cbuf):
    lanes = lax.iota(jnp.int32, NSEG)
    seg_base = lanes * SEGW

    for r0 in range(0, H, CHUNK):
        n = min(CHUNK, H - r0)
        pltpu.sync_copy(unc.at[b, pl.ds(r0, n)], staging.at[pl.ds(0, n)])

        def seg_row(r, _):
            rvec = jnp.full((NSEG,), r, jnp.int32)
            m = jnp.full((NSEG,), NEG, jnp.float32)
            for t in range(SEGW):
                v = plsc.load_gather(staging, [rvec, seg_base + t])
                m = jnp.maximum(m, v)
            rowseg[pl.ds((r0 + r) * NSEG, NSEG)] = m
            return 0

        lax.fori_loop(0, n, seg_row, 0)

    boxes = []
    cvacc = jnp.zeros((NSEG,), jnp.float32)

    for k in range(NPATCH):
        def row_scan(i, carry):
            m, ridx = carry
            v = rowseg[pl.ds(i * NSEG, NSEG)]
            upd = v > m
            ridx = jnp.where(upd, i, ridx)
            m = jnp.maximum(m, v)
            return m, ridx

        m0 = jnp.full((NSEG,), NEG, jnp.float32)
        i0 = jnp.zeros((NSEG,), jnp.int32)
        m, ridx = lax.fori_loop(0, H, row_scan, (m0, i0))
        g = jnp.max(m)
        rstar = jnp.min(jnp.where(m == g, ridx, BIGI))

        ra = (rstar // 8) * 8
        pltpu.sync_copy(unc.at[b, pl.ds(ra, 8)], staging.at[pl.ds(0, 8)])
        rloc = jnp.full((NSEG,), rstar - ra, jnp.int32)
        cmin = jnp.full((NSEG,), BIGI, jnp.int32)
        for t in range(W // NSEG):
            cvec = lanes + (t * NSEG)
            v = plsc.load_gather(staging, [rloc, cvec])
            for (bx1, by1, bx2, by2) in boxes:
                rowin = (rstar >= by1) & (rstar < by2)
                colin = (cvec >= bx1) & (cvec < bx2)
                v = jnp.where(colin & rowin, NEG, v)
            cmin = jnp.minimum(cmin, jnp.where(v == g, cvec, BIGI))
        cstar = jnp.min(cmin)

        x1 = jnp.clip(cstar - HALF, 0, W - EFF)
        y1 = jnp.clip(rstar - HALF, 0, H - EFF)

        for off, val in ((0, x1), (1, y1), (2, x1 + EFF), (3, y1 + EFF)):
            cvacc = jnp.where(lanes == (4 * k + off), val.astype(jnp.float32), cvacc)

        y1a = (y1 // 8) * 8
        dy = y1 - y1a
        for c in range(CHANS):
            pltpu.sync_copy(images.at[b, c, pl.ds(y1a, EFF + 8)],
                            staging.at[pl.ds(0, EFF + 8)])
            obase = (k * CHANS + c) * PSZ

            def patch_row(r, _):
                rvec = jnp.full((NSEG,), r + dy, jnp.int32)
                for t in range(EFF // NSEG):
                    v = plsc.load_gather(staging, [rvec, lanes + (x1 + t * NSEG)])
                    pbuf[0, pl.ds(obase + r * EFF + t * NSEG, NSEG)] = v
                return 0

            lax.fori_loop(0, EFF, patch_row, 0)

        if k < NPATCH - 1:
            mx1 = jnp.maximum(x1 - MARGIN, 0)
            my1 = jnp.maximum(y1 - MARGIN, 0)
            mx2 = jnp.minimum(x1 + EFF + MARGIN, W)
            my2 = jnp.minimum(y1 + EFF + MARGIN, H)
            boxes.append((mx1, my1, mx2, my2))
            jbase = jnp.minimum(mx1 // SEGW, NSEG - 4)
            rbase = (jnp.minimum(my1, H - SROWS) // 8) * 8
            pltpu.sync_copy(unc.at[b, pl.ds(rbase, SROWS)], staging)

            def mask_row(r, _):
                rvec = jnp.full((NSEG,), r - rbase, jnp.int32)
                rs = rowseg[pl.ds(r * NSEG, NSEG)]
                for dj in range(4):
                    j = jbase + dj
                    mm = jnp.full((NSEG,), NEG, jnp.float32)
                    for u in range(SEGW // NSEG):
                        cvec = lanes + (j * SEGW + u * NSEG)
                        v = plsc.load_gather(staging, [rvec, cvec])
                        for (bx1, by1, bx2, by2) in boxes:
                            rowin = (r >= by1) & (r < by2)
                            colin = (cvec >= bx1) & (cvec < bx2)
                            v = jnp.where(colin & rowin, NEG, v)
                        mm = jnp.maximum(mm, v)
                    s = jnp.max(mm)
                    rs = jnp.where(lanes == j, s, rs)
                rowseg[pl.ds(r * NSEG, NSEG)] = rs
                return 0

            lax.fori_loop(my1, my2, mask_row, 0)

    cbuf[0, pl.ds(0, NSEG)] = cvacc
    pltpu.sync_copy(pbuf, patches_out.at[b])
    pltpu.sync_copy(cbuf, coords_out.at[b])


def _sc_kernel(images, unc, patches_out, coords_out, staging, rowseg, pbuf, cbuf):
    cid = lax.axis_index("c")
    sid = lax.axis_index("s")
    wid = sid * 2 + cid

    @pl.when(wid < BATCH)
    def _():
        _worker(wid, images, unc, patches_out, coords_out, staging, rowseg,
                pbuf, cbuf)


@jax.jit
def kernel(images, uncertainty_maps):
    unc = uncertainty_maps.reshape(BATCH, H, W)
    mesh = plsc.VectorSubcoreMesh(
        core_axis_name="c", subcore_axis_name="s", num_cores=2, num_subcores=16)
    patches, coords_raw = pl.kernel(
        _sc_kernel,
        out_type=(
            jax.ShapeDtypeStruct((BATCH, 1, NPATCH * CHANS * PSZ), jnp.float32),
            jax.ShapeDtypeStruct((BATCH, 1, NSEG), jnp.float32),
        ),
        mesh=mesh,
        compiler_params=pltpu.CompilerParams(needs_layout_passes=False),
        scratch_types=[
            pltpu.VMEM((SROWS, W), jnp.float32),
            pltpu.VMEM((H * NSEG,), jnp.float32),
            pltpu.VMEM((1, NPATCH * CHANS * PSZ), jnp.float32),
            pltpu.VMEM((1, NSEG), jnp.float32),
        ],
    )(images, unc)
    patches = patches.reshape(BATCH, NPATCH, CHANS, EFF, EFF)
    coords = coords_raw[:, 0, :4 * NPATCH].reshape(BATCH, NPATCH, 4)
    return patches, coords


# speedup vs baseline: 2.3144x; 1.6435x over previous
import functools
import jax
import jax.numpy as jnp
from jax import lax
from jax.experimental import pallas as pl
from jax.experimental.pallas import tpu as pltpu
from jax.experimental.pallas import tpu_sc as plsc

H = 512
W = 512
EFF = 64
HALF = 32
MARGIN = 16
NPATCH = 3
CHANS = 3
BATCH = 16
NSEG = 16
BLK = 16
NBLK = H // BLK
NV = W // NSEG
SROWS = 128
PCHUNK = 128
PSZ = EFF * EFF
NEG = float(-3.4028235e38)
BIGI = 1 << 30


def _worker(b, images, unc, patches_out, coords_out, staging, colmax, pbuf, cbuf):
    lanes = lax.iota(jnp.int32, NSEG)

    def chunk_body(ck, _):
        pltpu.sync_copy(unc.at[b, pl.ds(ck * PCHUNK, PCHUNK)],
                        staging.at[pl.ds(0, PCHUNK)])

        def blk_body(lb, _):
            rvecs = [jnp.full((NSEG,), lb * BLK + rr, jnp.int32)
                     for rr in range(BLK)]
            blk = ck * (PCHUNK // BLK) + lb
            for i in range(NV):
                cvec = lanes + 16 * i
                acc = jnp.full((NSEG,), NEG, jnp.float32)
                for rr in range(BLK):
                    acc = jnp.maximum(
                        acc, plsc.load_gather(staging, [rvecs[rr], cvec]))
                colmax[pl.ds(blk * W + 16 * i, NSEG)] = acc
            return 0

        lax.fori_loop(0, PCHUNK // BLK, blk_body, 0)
        return 0

    lax.fori_loop(0, H // PCHUNK, chunk_body, 0)

    boxes = []
    cvacc = jnp.zeros((NSEG,), jnp.float32)

    for k in range(NPATCH):
        def scan_body(blk, carry):
            m, bidx = carry
            for i in range(NV):
                v = colmax[pl.ds(blk * W + 16 * i, NSEG)]
                upd = v > m
                bidx = jnp.where(upd, blk, bidx)
                m = jnp.maximum(m, v)
            return m, bidx

        m0 = jnp.full((NSEG,), NEG, jnp.float32)
        b0 = jnp.zeros((NSEG,), jnp.int32)
        m, bidx = lax.fori_loop(0, NBLK, scan_body, (m0, b0))
        g = jnp.max(m)
        bstar = jnp.min(jnp.where(m == g, bidx, BIGI))
        rb = bstar * BLK

        pltpu.sync_copy(unc.at[b, pl.ds(rb, BLK)], staging.at[pl.ds(0, BLK)])

        def find_body(r, carry):
            rbest, cbest = carry
            rvec = jnp.full((NSEG,), r, jnp.int32)
            cmin = jnp.full((NSEG,), BIGI, jnp.int32)
            rglob = rb + r
            for i in range(NV):
                cvec = lanes + 16 * i
                v = plsc.load_gather(staging, [rvec, cvec])
                for (bx1, by1, bx2, by2) in boxes:
                    rowin = (rglob >= by1) & (rglob < by2)
                    colin = (cvec >= bx1) & (cvec < bx2)
                    v = jnp.where(colin & rowin, NEG, v)
                cmin = jnp.minimum(cmin, jnp.where(v == g, cvec, BIGI))
            c_r = jnp.min(cmin)
            hit = (c_r < BIGI) & (rbest == BIGI)
            rbest = jnp.where(hit, rglob, rbest)
            cbest = jnp.where(hit, c_r, cbest)
            return rbest, cbest

        rstar, cstar = lax.fori_loop(
            0, BLK, find_body,
            (jnp.full((), BIGI, jnp.int32), jnp.full((), BIGI, jnp.int32)))

        x1 = jnp.clip(cstar - HALF, 0, W - EFF)
        y1 = jnp.clip(rstar - HALF, 0, H - EFF)

        for off, val in ((0, x1), (1, y1), (2, x1 + EFF), (3, y1 + EFF)):
            cvacc = jnp.where(lanes == (4 * k + off), val.astype(jnp.float32), cvacc)

        y1a = (y1 // 8) * 8
        dy = y1 - y1a
        for c in range(CHANS):
            pltpu.sync_copy(images.at[b, c, pl.ds(y1a, EFF + 8)],
                            staging.at[pl.ds(0, EFF + 8)])
            obase = (k * CHANS + c) * PSZ

            def patch_row(r, _):
                rvec = jnp.full((NSEG,), r + dy, jnp.int32)
                for t in range(EFF // NSEG):
                    v = plsc.load_gather(staging, [rvec, lanes + (x1 + t * NSEG)])
                    pbuf[0, pl.ds(obase + r * EFF + t * NSEG, NSEG)] = v
                return 0

            lax.fori_loop(0, EFF, patch_row, 0)

        if k < NPATCH - 1:
            mx1 = jnp.maximum(x1 - MARGIN, 0)
            my1 = jnp.maximum(y1 - MARGIN, 0)
            mx2 = jnp.minimum(x1 + EFF + MARGIN, W)
            my2 = jnp.minimum(y1 + EFF + MARGIN, H)
            boxes.append((mx1, my1, mx2, my2))
            ibase = jnp.minimum(mx1 // NSEG, NV - 7)
            bbase = jnp.minimum(my1 // BLK, NBLK - 7)
            rbase = bbase * BLK
            pltpu.sync_copy(unc.at[b, pl.ds(rbase, 7 * BLK)],
                            staging.at[pl.ds(0, 7 * BLK)])

            def supp_blk(tb, _):
                blk = bbase + tb

                def supp_vreg(ti, _):
                    i = ibase + ti
                    cvec = lanes + i * NSEG
                    acc = jnp.full((NSEG,), NEG, jnp.float32)
                    for rr in range(BLK):
                        rvec = jnp.full((NSEG,), tb * BLK + rr, jnp.int32)
                        v = plsc.load_gather(staging, [rvec, cvec])
                        rglob = rbase + tb * BLK + rr
                        for (bx1, by1, bx2, by2) in boxes:
                            rowin = (rglob >= by1) & (rglob < by2)
                            colin = (cvec >= bx1) & (cvec < bx2)
                            v = jnp.where(colin & rowin, NEG, v)
                        acc = jnp.maximum(acc, v)
                    colmax[pl.ds(blk * W + i * NSEG, NSEG)] = acc
                    return 0

                lax.fori_loop(0, 7, supp_vreg, 0)
                return 0

            lax.fori_loop(0, 7, supp_blk, 0)

    cbuf[0, pl.ds(0, NSEG)] = cvacc
    pltpu.sync_copy(pbuf, patches_out.at[b])
    pltpu.sync_copy(cbuf, coords_out.at[b])


def _sc_kernel(images, unc, patches_out, coords_out, staging, colmax, pbuf, cbuf):
    cid = lax.axis_index("c")
    sid = lax.axis_index("s")
    wid = sid * 2 + cid

    @pl.when(wid < BATCH)
    def _():
        _worker(wid, images, unc, patches_out, coords_out, staging, colmax,
                pbuf, cbuf)


@jax.jit
def kernel(images, uncertainty_maps):
    unc = uncertainty_maps.reshape(BATCH, H, W)
    mesh = plsc.VectorSubcoreMesh(
        core_axis_name="c", subcore_axis_name="s", num_cores=2, num_subcores=16)
    patches, coords_raw = pl.kernel(
        _sc_kernel,
        out_type=(
            jax.ShapeDtypeStruct((BATCH, 1, NPATCH * CHANS * PSZ), jnp.float32),
            jax.ShapeDtypeStruct((BATCH, 1, NSEG), jnp.float32),
        ),
        mesh=mesh,
        compiler_params=pltpu.CompilerParams(needs_layout_passes=False),
        scratch_types=[
            pltpu.VMEM((SROWS, W), jnp.float32),
            pltpu.VMEM((NBLK * W,), jnp.float32),
            pltpu.VMEM((1, NPATCH * CHANS * PSZ), jnp.float32),
            pltpu.VMEM((1, NSEG), jnp.float32),
        ],
    )(images, unc)
    patches = patches.reshape(BATCH, NPATCH, CHANS, EFF, EFF)
    coords = coords_raw[:, 0, :4 * NPATCH].reshape(BATCH, NPATCH, 4)
    return patches, coords


# speedup vs baseline: 2.3917x; 1.0334x over previous
import functools
import jax
import jax.numpy as jnp
from jax import lax
from jax.experimental import pallas as pl
from jax.experimental.pallas import tpu as pltpu
from jax.experimental.pallas import tpu_sc as plsc

H = 512
W = 512
EFF = 64
HALF = 32
MARGIN = 16
NPATCH = 3
CHANS = 3
BATCH = 16
NSEG = 16
BLK = 16
NBLK = H // BLK
NV = W // NSEG
SROWS = 128
PCHUNK = 128
PSZ = EFF * EFF
NEG = float(-3.4028235e38)
BIGI = 1 << 30


def _worker(b, images, unc, patches_out, coords_out, staging, colmax, pbuf,
            cbuf, sem0, sem1):
    lanes = lax.iota(jnp.int32, NSEG)

    def make_blocks(lblk0, roff):
        def blk_body(lb, _):
            rvecs = [jnp.full((NSEG,), roff + lb * BLK + rr, jnp.int32)
                     for rr in range(BLK)]
            blk = lblk0 + lb
            for i in range(NV):
                cvec = lanes + 16 * i
                acc = jnp.full((NSEG,), NEG, jnp.float32)
                for rr in range(BLK):
                    acc = jnp.maximum(
                        acc, plsc.load_gather(staging, [rvecs[rr], cvec]))
                colmax[pl.ds(blk * W + 16 * i, NSEG)] = acc
            return 0

        lax.fori_loop(0, 4, blk_body, 0)

    pltpu.async_copy(unc.at[b, pl.ds(0, 64)], staging.at[pl.ds(0, 64)], sem0)
    pltpu.async_copy(unc.at[b, pl.ds(64, 64)], staging.at[pl.ds(64, 64)], sem1)

    def super_body(j, _):
        pltpu.make_async_copy(unc.at[b, pl.ds(0, 64)],
                              staging.at[pl.ds(0, 64)], sem0).wait()
        make_blocks(8 * j, 0)

        @pl.when(j < 3)
        def _():
            pltpu.async_copy(unc.at[b, pl.ds((2 * j + 2) * 64, 64)],
                             staging.at[pl.ds(0, 64)], sem0)

        pltpu.make_async_copy(unc.at[b, pl.ds(64, 64)],
                              staging.at[pl.ds(64, 64)], sem1).wait()
        make_blocks(8 * j + 4, 64)

        @pl.when(j < 3)
        def _():
            pltpu.async_copy(unc.at[b, pl.ds((2 * j + 3) * 64, 64)],
                             staging.at[pl.ds(64, 64)], sem1)

        return 0

    lax.fori_loop(0, 4, super_body, 0)

    boxes = []
    cvacc = jnp.zeros((NSEG,), jnp.float32)

    for k in range(NPATCH):
        def scan_body(blk, carry):
            m, bidx = carry
            for i in range(NV):
                v = colmax[pl.ds(blk * W + 16 * i, NSEG)]
                upd = v > m
                bidx = jnp.where(upd, blk, bidx)
                m = jnp.maximum(m, v)
            return m, bidx

        m0 = jnp.full((NSEG,), NEG, jnp.float32)
        b0 = jnp.zeros((NSEG,), jnp.int32)
        m, bidx = lax.fori_loop(0, NBLK, scan_body, (m0, b0))
        g = jnp.max(m)
        bstar = jnp.min(jnp.where(m == g, bidx, BIGI))
        rb = bstar * BLK

        pltpu.sync_copy(unc.at[b, pl.ds(rb, BLK)], staging.at[pl.ds(0, BLK)])

        def find_body(r, carry):
            rbest, cbest = carry
            rvec = jnp.full((NSEG,), r, jnp.int32)
            cmin = jnp.full((NSEG,), BIGI, jnp.int32)
            rglob = rb + r
            for i in range(NV):
                cvec = lanes + 16 * i
                v = plsc.load_gather(staging, [rvec, cvec])
                for (bx1, by1, bx2, by2) in boxes:
                    rowin = (rglob >= by1) & (rglob < by2)
                    colin = (cvec >= bx1) & (cvec < bx2)
                    v = jnp.where(colin & rowin, NEG, v)
                cmin = jnp.minimum(cmin, jnp.where(v == g, cvec, BIGI))
            c_r = jnp.min(cmin)
            hit = (c_r < BIGI) & (rbest == BIGI)
            rbest = jnp.where(hit, rglob, rbest)
            cbest = jnp.where(hit, c_r, cbest)
            return rbest, cbest

        rstar, cstar = lax.fori_loop(
            0, BLK, find_body,
            (jnp.full((), BIGI, jnp.int32), jnp.full((), BIGI, jnp.int32)))

        x1 = jnp.clip(cstar - HALF, 0, W - EFF)
        y1 = jnp.clip(rstar - HALF, 0, H - EFF)

        for off, val in ((0, x1), (1, y1), (2, x1 + EFF), (3, y1 + EFF)):
            cvacc = jnp.where(lanes == (4 * k + off), val.astype(jnp.float32), cvacc)

        y1a = (y1 // 8) * 8
        dy = y1 - y1a
        for c in range(CHANS):
            pltpu.sync_copy(images.at[b, c, pl.ds(y1a, EFF + 8)],
                            staging.at[pl.ds(0, EFF + 8)])
            obase = (k * CHANS + c) * PSZ

            def patch_row(r, _):
                rvec = jnp.full((NSEG,), r + dy, jnp.int32)
                for t in range(EFF // NSEG):
                    v = plsc.load_gather(staging, [rvec, lanes + (x1 + t * NSEG)])
                    pbuf[0, pl.ds(obase + r * EFF + t * NSEG, NSEG)] = v
                return 0

            lax.fori_loop(0, EFF, patch_row, 0)

        if k < NPATCH - 1:
            mx1 = jnp.maximum(x1 - MARGIN, 0)
            my1 = jnp.maximum(y1 - MARGIN, 0)
            mx2 = jnp.minimum(x1 + EFF + MARGIN, W)
            my2 = jnp.minimum(y1 + EFF + MARGIN, H)
            boxes.append((mx1, my1, mx2, my2))
            ibase = jnp.minimum(mx1 // NSEG, NV - 7)
            bbase = jnp.minimum(my1 // BLK, NBLK - 7)
            rbase = bbase * BLK
            pltpu.sync_copy(unc.at[b, pl.ds(rbase, 7 * BLK)],
                            staging.at[pl.ds(0, 7 * BLK)])

            def supp_blk(tb, _):
                blk = bbase + tb

                def supp_vreg(ti, _):
                    i = ibase + ti
                    cvec = lanes + i * NSEG
                    acc = jnp.full((NSEG,), NEG, jnp.float32)
                    for rr in range(BLK):
                        rvec = jnp.full((NSEG,), tb * BLK + rr, jnp.int32)
                        v = plsc.load_gather(staging, [rvec, cvec])
                        rglob = rbase + tb * BLK + rr
                        for (bx1, by1, bx2, by2) in boxes:
                            rowin = (rglob >= by1) & (rglob < by2)
                            colin = (cvec >= bx1) & (cvec < bx2)
                            v = jnp.where(colin & rowin, NEG, v)
                        acc = jnp.maximum(acc, v)
                    colmax[pl.ds(blk * W + i * NSEG, NSEG)] = acc
                    return 0

                lax.fori_loop(0, 7, supp_vreg, 0)
                return 0

            lax.fori_loop(0, 7, supp_blk, 0)

    cbuf[0, pl.ds(0, NSEG)] = cvacc
    pltpu.sync_copy(pbuf, patches_out.at[b])
    pltpu.sync_copy(cbuf, coords_out.at[b])


def _sc_kernel(images, unc, patches_out, coords_out, staging, colmax, pbuf,
               cbuf, sem0, sem1):
    cid = lax.axis_index("c")
    sid = lax.axis_index("s")
    wid = sid * 2 + cid

    @pl.when(wid < BATCH)
    def _():
        _worker(wid, images, unc, patches_out, coords_out, staging, colmax,
                pbuf, cbuf, sem0, sem1)


@jax.jit
def kernel(images, uncertainty_maps):
    unc = uncertainty_maps.reshape(BATCH, H, W)
    mesh = plsc.VectorSubcoreMesh(
        core_axis_name="c", subcore_axis_name="s", num_cores=2, num_subcores=16)
    patches, coords_raw = pl.kernel(
        _sc_kernel,
        out_type=(
            jax.ShapeDtypeStruct((BATCH, 1, NPATCH * CHANS * PSZ), jnp.float32),
            jax.ShapeDtypeStruct((BATCH, 1, NSEG), jnp.float32),
        ),
        mesh=mesh,
        compiler_params=pltpu.CompilerParams(needs_layout_passes=False),
        scratch_types=[
            pltpu.VMEM((SROWS, W), jnp.float32),
            pltpu.VMEM((NBLK * W,), jnp.float32),
            pltpu.VMEM((1, NPATCH * CHANS * PSZ), jnp.float32),
            pltpu.VMEM((1, NSEG), jnp.float32),
            pltpu.SemaphoreType.DMA,
            pltpu.SemaphoreType.DMA,
        ],
    )(images, unc)
    patches = patches.reshape(BATCH, NPATCH, CHANS, EFF, EFF)
    coords = coords_raw[:, 0, :4 * NPATCH].reshape(BATCH, NPATCH, 4)
    return patches, coords


# speedup vs baseline: 2.6294x; 1.0994x over previous
import functools
import jax
import jax.numpy as jnp
from jax import lax
from jax.experimental import pallas as pl
from jax.experimental.pallas import tpu as pltpu
from jax.experimental.pallas import tpu_sc as plsc

H = 512
W = 512
EFF = 64
HALF = 32
MARGIN = 16
NPATCH = 3
CHANS = 3
BATCH = 16
NSEG = 16
BLK = 16
NBLK = H // BLK
NV = W // NSEG
SROWS = 128
PCHUNK = 128
PSZ = EFF * EFF
NEG = float(-3.4028235e38)
BIGI = 1 << 30


def _worker(b, images, unc, patches_out, coords_out, staging, colmax, pbuf,
            cbuf, sem0, sem1):
    lanes = lax.iota(jnp.int32, NSEG)

    def make_blocks(lblk0, roff):
        def blk_body(lb, _):
            rvecs = [jnp.full((NSEG,), roff + lb * BLK + rr, jnp.int32)
                     for rr in range(BLK)]
            blk = lblk0 + lb
            for i in range(NV):
                cvec = lanes + 16 * i
                acc = jnp.full((NSEG,), NEG, jnp.float32)
                for rr in range(BLK):
                    acc = jnp.maximum(
                        acc, plsc.load_gather(staging, [rvecs[rr], cvec]))
                colmax[pl.ds(blk * W + 16 * i, NSEG)] = acc
            return 0

        lax.fori_loop(0, 4, blk_body, 0)

    pltpu.async_copy(unc.at[b, pl.ds(0, 64)], staging.at[pl.ds(0, 64)], sem0)
    pltpu.async_copy(unc.at[b, pl.ds(64, 64)], staging.at[pl.ds(64, 64)], sem1)

    def super_body(j, _):
        pltpu.make_async_copy(unc.at[b, pl.ds(0, 64)],
                              staging.at[pl.ds(0, 64)], sem0).wait()
        make_blocks(8 * j, 0)

        @pl.when(j < 3)
        def _():
            pltpu.async_copy(unc.at[b, pl.ds((2 * j + 2) * 64, 64)],
                             staging.at[pl.ds(0, 64)], sem0)

        pltpu.make_async_copy(unc.at[b, pl.ds(64, 64)],
                              staging.at[pl.ds(64, 64)], sem1).wait()
        make_blocks(8 * j + 4, 64)

        @pl.when(j < 3)
        def _():
            pltpu.async_copy(unc.at[b, pl.ds((2 * j + 3) * 64, 64)],
                             staging.at[pl.ds(64, 64)], sem1)

        return 0

    lax.fori_loop(0, 4, super_body, 0)

    boxes = []
    cvacc = jnp.zeros((NSEG,), jnp.float32)

    for k in range(NPATCH):
        def scan_body(blk, carry):
            m, bidx = carry
            for i in range(NV):
                v = colmax[pl.ds(blk * W + 16 * i, NSEG)]
                upd = v > m
                bidx = jnp.where(upd, blk, bidx)
                m = jnp.maximum(m, v)
            return m, bidx

        m0 = jnp.full((NSEG,), NEG, jnp.float32)
        b0 = jnp.zeros((NSEG,), jnp.int32)
        m, bidx = lax.fori_loop(0, NBLK, scan_body, (m0, b0))
        g = jnp.max(m)
        bstar = jnp.min(jnp.where(m == g, bidx, BIGI))
        rb = bstar * BLK

        pltpu.sync_copy(unc.at[b, pl.ds(rb, BLK)], staging.at[pl.ds(0, BLK)])

        def find_body(r, carry):
            rbest, cbest = carry
            rvec = jnp.full((NSEG,), r, jnp.int32)
            cmin = jnp.full((NSEG,), BIGI, jnp.int32)
            rglob = rb + r
            for i in range(NV):
                cvec = lanes + 16 * i
                v = plsc.load_gather(staging, [rvec, cvec])
                for (bx1, by1, bx2, by2) in boxes:
                    rowin = (rglob >= by1) & (rglob < by2)
                    colin = (cvec >= bx1) & (cvec < bx2)
                    v = jnp.where(colin & rowin, NEG, v)
                cmin = jnp.minimum(cmin, jnp.where(v == g, cvec, BIGI))
            c_r = jnp.min(cmin)
            hit = (c_r < BIGI) & (rbest == BIGI)
            rbest = jnp.where(hit, rglob, rbest)
            cbest = jnp.where(hit, c_r, cbest)
            return rbest, cbest

        rstar, cstar = lax.fori_loop(
            0, BLK, find_body,
            (jnp.full((), BIGI, jnp.int32), jnp.full((), BIGI, jnp.int32)))

        x1 = jnp.clip(cstar - HALF, 0, W - EFF)
        y1 = jnp.clip(rstar - HALF, 0, H - EFF)

        for off, val in ((0, x1), (1, y1), (2, x1 + EFF), (3, y1 + EFF)):
            cvacc = jnp.where(lanes == (4 * k + off), val.astype(jnp.float32), cvacc)

        y1a = (y1 // 8) * 8
        dy = y1 - y1a
        x1b = jnp.minimum((x1 // 128) * 128, W - 256)
        dx = x1 - x1b
        hs = [
            pltpu.async_copy(images.at[b, 0, pl.ds(y1a, EFF + 8), pl.ds(x1b, 256)],
                             staging.at[pl.ds(0, EFF + 8), pl.ds(0, 256)], sem0),
            pltpu.async_copy(images.at[b, 1, pl.ds(y1a, EFF + 8), pl.ds(x1b, 256)],
                             staging.at[pl.ds(0, EFF + 8), pl.ds(256, 256)], sem1),
            None,
        ]
        for c in range(CHANS):
            hs[c].wait()
            obase = (k * CHANS + c) * PSZ
            coff = dx + (256 if c == 1 else 0)

            def patch_row(r, _):
                rvec = jnp.full((NSEG,), r + dy, jnp.int32)
                for t in range(EFF // NSEG):
                    v = plsc.load_gather(staging, [rvec, lanes + (coff + t * NSEG)])
                    pbuf[0, pl.ds(obase + r * EFF + t * NSEG, NSEG)] = v
                return 0

            lax.fori_loop(0, EFF, patch_row, 0)
            if c == 0:
                hs[2] = pltpu.async_copy(
                    images.at[b, 2, pl.ds(y1a, EFF + 8), pl.ds(x1b, 256)],
                    staging.at[pl.ds(0, EFF + 8), pl.ds(0, 256)], sem0)

        if k < NPATCH - 1:
            mx1 = jnp.maximum(x1 - MARGIN, 0)
            my1 = jnp.maximum(y1 - MARGIN, 0)
            mx2 = jnp.minimum(x1 + EFF + MARGIN, W)
            my2 = jnp.minimum(y1 + EFF + MARGIN, H)
            boxes.append((mx1, my1, mx2, my2))
            ibase = jnp.minimum(mx1 // NSEG, NV - 7)
            bbase = jnp.minimum(my1 // BLK, NBLK - 7)
            rbase = bbase * BLK
            pltpu.sync_copy(unc.at[b, pl.ds(rbase, 7 * BLK)],
                            staging.at[pl.ds(0, 7 * BLK)])

            def supp_blk(tb, _):
                blk = bbase + tb

                def supp_vreg(ti, _):
                    i = ibase + ti
                    cvec = lanes + i * NSEG
                    acc = jnp.full((NSEG,), NEG, jnp.float32)
                    for rr in range(BLK):
                        rvec = jnp.full((NSEG,), tb * BLK + rr, jnp.int32)
                        v = plsc.load_gather(staging, [rvec, cvec])
                        rglob = rbase + tb * BLK + rr
                        for (bx1, by1, bx2, by2) in boxes:
                            rowin = (rglob >= by1) & (rglob < by2)
                            colin = (cvec >= bx1) & (cvec < bx2)
                            v = jnp.where(colin & rowin, NEG, v)
                        acc = jnp.maximum(acc, v)
                    colmax[pl.ds(blk * W + i * NSEG, NSEG)] = acc
                    return 0

                lax.fori_loop(0, 7, supp_vreg, 0)
                return 0

            lax.fori_loop(0, 7, supp_blk, 0)

    cbuf[0, pl.ds(0, NSEG)] = cvacc
    pltpu.sync_copy(pbuf, patches_out.at[b])
    pltpu.sync_copy(cbuf, coords_out.at[b])


def _sc_kernel(images, unc, patches_out, coords_out, staging, colmax, pbuf,
               cbuf, sem0, sem1):
    cid = lax.axis_index("c")
    sid = lax.axis_index("s")
    wid = sid * 2 + cid

    @pl.when(wid < BATCH)
    def _():
        _worker(wid, images, unc, patches_out, coords_out, staging, colmax,
                pbuf, cbuf, sem0, sem1)


@jax.jit
def kernel(images, uncertainty_maps):
    unc = uncertainty_maps.reshape(BATCH, H, W)
    mesh = plsc.VectorSubcoreMesh(
        core_axis_name="c", subcore_axis_name="s", num_cores=2, num_subcores=16)
    patches, coords_raw = pl.kernel(
        _sc_kernel,
        out_type=(
            jax.ShapeDtypeStruct((BATCH, 1, NPATCH * CHANS * PSZ), jnp.float32),
            jax.ShapeDtypeStruct((BATCH, 1, NSEG), jnp.float32),
        ),
        mesh=mesh,
        compiler_params=pltpu.CompilerParams(needs_layout_passes=False),
        scratch_types=[
            pltpu.VMEM((SROWS, W), jnp.float32),
            pltpu.VMEM((NBLK * W,), jnp.float32),
            pltpu.VMEM((1, NPATCH * CHANS * PSZ), jnp.float32),
            pltpu.VMEM((1, NSEG), jnp.float32),
            pltpu.SemaphoreType.DMA,
            pltpu.SemaphoreType.DMA,
        ],
    )(images, unc)
    patches = patches.reshape(BATCH, NPATCH, CHANS, EFF, EFF)
    coords = coords_raw[:, 0, :4 * NPATCH].reshape(BATCH, NPATCH, 4)
    return patches, coords


# speedup vs baseline: 2.6715x; 1.0160x over previous
import functools
import jax
import jax.numpy as jnp
from jax import lax
from jax.experimental import pallas as pl
from jax.experimental.pallas import tpu as pltpu
from jax.experimental.pallas import tpu_sc as plsc

H = 512
W = 512
EFF = 64
HALF = 32
MARGIN = 16
NPATCH = 3
CHANS = 3
BATCH = 16
NSEG = 16
BLK = 16
NBLK = H // BLK
NV = W // NSEG
SROWS = 128
PCHUNK = 128
PSZ = EFF * EFF
NEG = float(-3.4028235e38)
BIGI = 1 << 30


def _worker(b, images, unc, patches_out, coords_out, staging, colmax, summ,
            pbuf, cbuf, sem0, sem1):
    lanes = lax.iota(jnp.int32, NSEG)

    def make_blocks(lblk0, roff):
        def blk_body(lb, _):
            rvecs = [jnp.full((NSEG,), roff + lb * BLK + rr, jnp.int32)
                     for rr in range(BLK)]
            blk = lblk0 + lb
            for i in range(NV):
                cvec = lanes + 16 * i
                acc = jnp.full((NSEG,), NEG, jnp.float32)
                for rr in range(BLK):
                    acc = jnp.maximum(
                        acc, plsc.load_gather(staging, [rvecs[rr], cvec]))
                colmax[pl.ds(blk * W + 16 * i, NSEG)] = acc
                s = acc if i == 0 else jnp.maximum(s, acc)
            summ[pl.ds(blk * NSEG, NSEG)] = s
            return 0

        lax.fori_loop(0, 4, blk_body, 0)

    pltpu.async_copy(unc.at[b, pl.ds(0, 64)], staging.at[pl.ds(0, 64)], sem0)
    pltpu.async_copy(unc.at[b, pl.ds(64, 64)], staging.at[pl.ds(64, 64)], sem1)

    def super_body(j, _):
        pltpu.make_async_copy(unc.at[b, pl.ds(0, 64)],
                              staging.at[pl.ds(0, 64)], sem0).wait()
        make_blocks(8 * j, 0)

        @pl.when(j < 3)
        def _():
            pltpu.async_copy(unc.at[b, pl.ds((2 * j + 2) * 64, 64)],
                             staging.at[pl.ds(0, 64)], sem0)

        pltpu.make_async_copy(unc.at[b, pl.ds(64, 64)],
                              staging.at[pl.ds(64, 64)], sem1).wait()
        make_blocks(8 * j + 4, 64)

        @pl.when(j < 3)
        def _():
            pltpu.async_copy(unc.at[b, pl.ds((2 * j + 3) * 64, 64)],
                             staging.at[pl.ds(64, 64)], sem1)

        return 0

    lax.fori_loop(0, 4, super_body, 0)

    boxes = []
    cvacc = jnp.zeros((NSEG,), jnp.float32)

    for k in range(NPATCH):
        m = jnp.full((NSEG,), NEG, jnp.float32)
        bidx = jnp.zeros((NSEG,), jnp.int32)
        for blk in range(NBLK):
            v = summ[pl.ds(blk * NSEG, NSEG)]
            upd = v > m
            bidx = jnp.where(upd, blk, bidx)
            m = jnp.maximum(m, v)
        g = jnp.max(m)
        bstar = jnp.min(jnp.where(m == g, bidx, BIGI))
        rb = bstar * BLK

        pltpu.sync_copy(unc.at[b, pl.ds(rb, BLK)], staging.at[pl.ds(0, BLK)])

        def find_body(r, carry):
            rbest, cbest = carry
            rvec = jnp.full((NSEG,), r, jnp.int32)
            cmin = jnp.full((NSEG,), BIGI, jnp.int32)
            rglob = rb + r
            for i in range(NV):
                cvec = lanes + 16 * i
                v = plsc.load_gather(staging, [rvec, cvec])
                for (bx1, by1, bx2, by2) in boxes:
                    rowin = (rglob >= by1) & (rglob < by2)
                    colin = (cvec >= bx1) & (cvec < bx2)
                    v = jnp.where(colin & rowin, NEG, v)
                cmin = jnp.minimum(cmin, jnp.where(v == g, cvec, BIGI))
            c_r = jnp.min(cmin)
            hit = (c_r < BIGI) & (rbest == BIGI)
            rbest = jnp.where(hit, rglob, rbest)
            cbest = jnp.where(hit, c_r, cbest)
            return rbest, cbest

        rstar, cstar = lax.fori_loop(
            0, BLK, find_body,
            (jnp.full((), BIGI, jnp.int32), jnp.full((), BIGI, jnp.int32)))

        x1 = jnp.clip(cstar - HALF, 0, W - EFF)
        y1 = jnp.clip(rstar - HALF, 0, H - EFF)

        for off, val in ((0, x1), (1, y1), (2, x1 + EFF), (3, y1 + EFF)):
            cvacc = jnp.where(lanes == (4 * k + off), val.astype(jnp.float32), cvacc)

        y1a = (y1 // 8) * 8
        dy = y1 - y1a
        x1b = jnp.minimum((x1 // 128) * 128, W - 256)
        dx = x1 - x1b
        hs = [
            pltpu.async_copy(images.at[b, 0, pl.ds(y1a, EFF + 8), pl.ds(x1b, 256)],
                             staging.at[pl.ds(0, EFF + 8), pl.ds(0, 256)], sem0),
            pltpu.async_copy(images.at[b, 1, pl.ds(y1a, EFF + 8), pl.ds(x1b, 256)],
                             staging.at[pl.ds(0, EFF + 8), pl.ds(256, 256)], sem1),
            None,
        ]
        for c in range(CHANS):
            hs[c].wait()
            obase = (k * CHANS + c) * PSZ
            coff = dx + (256 if c == 1 else 0)

            def patch_row(r, _):
                rvec = jnp.full((NSEG,), r + dy, jnp.int32)
                for t in range(EFF // NSEG):
                    v = plsc.load_gather(staging, [rvec, lanes + (coff + t * NSEG)])
                    pbuf[0, pl.ds(obase + r * EFF + t * NSEG, NSEG)] = v
                return 0

            lax.fori_loop(0, EFF, patch_row, 0)
            if c == 0:
                hs[2] = pltpu.async_copy(
                    images.at[b, 2, pl.ds(y1a, EFF + 8), pl.ds(x1b, 256)],
                    staging.at[pl.ds(0, EFF + 8), pl.ds(0, 256)], sem0)

        if k < NPATCH - 1:
            mx1 = jnp.maximum(x1 - MARGIN, 0)
            my1 = jnp.maximum(y1 - MARGIN, 0)
            mx2 = jnp.minimum(x1 + EFF + MARGIN, W)
            my2 = jnp.minimum(y1 + EFF + MARGIN, H)
            boxes.append((mx1, my1, mx2, my2))
            ibase = jnp.minimum(mx1 // NSEG, NV - 7)
            bbase = jnp.minimum(my1 // BLK, NBLK - 7)
            rbase = bbase * BLK
            pltpu.sync_copy(unc.at[b, pl.ds(rbase, 7 * BLK)],
                            staging.at[pl.ds(0, 7 * BLK)])

            def supp_blk(tb, _):
                blk = bbase + tb

                def supp_vreg(ti, _):
                    i = ibase + ti
                    cvec = lanes + i * NSEG
                    acc = jnp.full((NSEG,), NEG, jnp.float32)
                    for rr in range(BLK):
                        rvec = jnp.full((NSEG,), tb * BLK + rr, jnp.int32)
                        v = plsc.load_gather(staging, [rvec, cvec])
                        rglob = rbase + tb * BLK + rr
                        for (bx1, by1, bx2, by2) in boxes:
                            rowin = (rglob >= by1) & (rglob < by2)
                            colin = (cvec >= bx1) & (cvec < bx2)
                            v = jnp.where(colin & rowin, NEG, v)
                        acc = jnp.maximum(acc, v)
                    colmax[pl.ds(blk * W + i * NSEG, NSEG)] = acc
                    return 0

                lax.fori_loop(0, 7, supp_vreg, 0)
                for i in range(NV):
                    v = colmax[pl.ds(blk * W + i * NSEG, NSEG)]
                    s = v if i == 0 else jnp.maximum(s, v)
                summ[pl.ds(blk * NSEG, NSEG)] = s
                return 0

            lax.fori_loop(0, 7, supp_blk, 0)

    cbuf[0, pl.ds(0, NSEG)] = cvacc
    pltpu.sync_copy(pbuf, patches_out.at[b])
    pltpu.sync_copy(cbuf, coords_out.at[b])


def _sc_kernel(images, unc, patches_out, coords_out, staging, colmax, summ,
               pbuf, cbuf, sem0, sem1):
    cid = lax.axis_index("c")
    sid = lax.axis_index("s")
    wid = sid * 2 + cid

    @pl.when(wid < BATCH)
    def _():
        _worker(wid, images, unc, patches_out, coords_out, staging, colmax,
                summ, pbuf, cbuf, sem0, sem1)


@jax.jit
def kernel(images, uncertainty_maps):
    unc = uncertainty_maps.reshape(BATCH, H, W)
    mesh = plsc.VectorSubcoreMesh(
        core_axis_name="c", subcore_axis_name="s", num_cores=2, num_subcores=16)
    patches, coords_raw = pl.kernel(
        _sc_kernel,
        out_type=(
            jax.ShapeDtypeStruct((BATCH, 1, NPATCH * CHANS * PSZ), jnp.float32),
            jax.ShapeDtypeStruct((BATCH, 1, NSEG), jnp.float32),
        ),
        mesh=mesh,
        compiler_params=pltpu.CompilerParams(needs_layout_passes=False),
        scratch_types=[
            pltpu.VMEM((SROWS, W), jnp.float32),
            pltpu.VMEM((NBLK * W,), jnp.float32),
            pltpu.VMEM((NBLK * NSEG,), jnp.float32),
            pltpu.VMEM((1, NPATCH * CHANS * PSZ), jnp.float32),
            pltpu.VMEM((1, NSEG), jnp.float32),
            pltpu.SemaphoreType.DMA,
            pltpu.SemaphoreType.DMA,
        ],
    )(images, unc)
    patches = patches.reshape(BATCH, NPATCH, CHANS, EFF, EFF)
    coords = coords_raw[:, 0, :4 * NPATCH].reshape(BATCH, NPATCH, 4)
    return patches, coords


# speedup vs baseline: 3.0182x; 1.1298x over previous
import functools
import jax
import jax.numpy as jnp
from jax import lax
from jax.experimental import pallas as pl
from jax.experimental.pallas import tpu as pltpu
from jax.experimental.pallas import tpu_sc as plsc

H = 512
W = 512
EFF = 64
HALF = 32
MARGIN = 16
NPATCH = 3
CHANS = 3
BATCH = 16
NSEG = 16
BLK = 8
NBLK = H // BLK
NV = W // NSEG
SROWS = 72
PCHUNK = 128
PSZ = EFF * EFF
NEG = float(-3.4028235e38)
BIGI = 1 << 30


def _worker(b, images, unc, patches_out, coords_out, staging, colmax, summ,
            pbuf, cbuf, sem0, sem1):
    lanes = lax.iota(jnp.int32, NSEG)

    def make_blocks(lblk0, roff):
        def blk_body(lb, _):
            rvecs = [jnp.full((NSEG,), roff + lb * BLK + rr, jnp.int32)
                     for rr in range(BLK)]
            blk = lblk0 + lb
            for i in range(NV):
                cvec = lanes + 16 * i
                acc = jnp.full((NSEG,), NEG, jnp.float32)
                for rr in range(BLK):
                    acc = jnp.maximum(
                        acc, plsc.load_gather(staging, [rvecs[rr], cvec]))
                colmax[pl.ds(blk * W + 16 * i, NSEG)] = acc
                s = acc if i == 0 else jnp.maximum(s, acc)
            summ[pl.ds(blk * NSEG, NSEG)] = s
            return 0

        lax.fori_loop(0, 4, blk_body, 0)

    pltpu.async_copy(unc.at[b, pl.ds(0, 32)], staging.at[pl.ds(0, 32)], sem0)
    pltpu.async_copy(unc.at[b, pl.ds(32, 32)], staging.at[pl.ds(32, 32)], sem1)

    def super_body(j, _):
        pltpu.make_async_copy(unc.at[b, pl.ds(0, 32)],
                              staging.at[pl.ds(0, 32)], sem0).wait()
        make_blocks(8 * j, 0)

        @pl.when(j < 7)
        def _():
            pltpu.async_copy(unc.at[b, pl.ds((2 * j + 2) * 32, 32)],
                             staging.at[pl.ds(0, 32)], sem0)

        pltpu.make_async_copy(unc.at[b, pl.ds(32, 32)],
                              staging.at[pl.ds(32, 32)], sem1).wait()
        make_blocks(8 * j + 4, 32)

        @pl.when(j < 7)
        def _():
            pltpu.async_copy(unc.at[b, pl.ds((2 * j + 3) * 32, 32)],
                             staging.at[pl.ds(32, 32)], sem1)

        return 0

    lax.fori_loop(0, 8, super_body, 0)

    boxes = []
    cvacc = jnp.zeros((NSEG,), jnp.float32)

    for k in range(NPATCH):
        m = jnp.full((NSEG,), NEG, jnp.float32)
        bidx = jnp.zeros((NSEG,), jnp.int32)
        for blk in range(NBLK):
            v = summ[pl.ds(blk * NSEG, NSEG)]
            upd = v > m
            bidx = jnp.where(upd, blk, bidx)
            m = jnp.maximum(m, v)
        g = jnp.max(m)
        bstar = jnp.min(jnp.where(m == g, bidx, BIGI))
        rb = bstar * BLK

        pltpu.sync_copy(unc.at[b, pl.ds(rb, BLK)], staging.at[pl.ds(0, BLK)])

        def find_body(r, carry):
            rbest, cbest = carry
            rvec = jnp.full((NSEG,), r, jnp.int32)
            cmin = jnp.full((NSEG,), BIGI, jnp.int32)
            rglob = rb + r
            for i in range(NV):
                cvec = lanes + 16 * i
                v = plsc.load_gather(staging, [rvec, cvec])
                for (bx1, by1, bx2, by2) in boxes:
                    rowin = (rglob >= by1) & (rglob < by2)
                    colin = (cvec >= bx1) & (cvec < bx2)
                    v = jnp.where(colin & rowin, NEG, v)
                cmin = jnp.minimum(cmin, jnp.where(v == g, cvec, BIGI))
            c_r = jnp.min(cmin)
            hit = (c_r < BIGI) & (rbest == BIGI)
            rbest = jnp.where(hit, rglob, rbest)
            cbest = jnp.where(hit, c_r, cbest)
            return rbest, cbest

        rstar, cstar = lax.fori_loop(
            0, BLK, find_body,
            (jnp.full((), BIGI, jnp.int32), jnp.full((), BIGI, jnp.int32)))

        x1 = jnp.clip(cstar - HALF, 0, W - EFF)
        y1 = jnp.clip(rstar - HALF, 0, H - EFF)

        for off, val in ((0, x1), (1, y1), (2, x1 + EFF), (3, y1 + EFF)):
            cvacc = jnp.where(lanes == (4 * k + off), val.astype(jnp.float32), cvacc)

        y1a = (y1 // 8) * 8
        dy = y1 - y1a
        x1b = jnp.minimum((x1 // 128) * 128, W - 256)
        dx = x1 - x1b
        hs = [
            pltpu.async_copy(images.at[b, 0, pl.ds(y1a, EFF + 8), pl.ds(x1b, 256)],
                             staging.at[pl.ds(0, EFF + 8), pl.ds(0, 256)], sem0),
            pltpu.async_copy(images.at[b, 1, pl.ds(y1a, EFF + 8), pl.ds(x1b, 256)],
                             staging.at[pl.ds(0, EFF + 8), pl.ds(256, 256)], sem1),
            None,
        ]
        for c in range(CHANS):
            hs[c].wait()
            obase = (k * CHANS + c) * PSZ
            coff = dx + (256 if c == 1 else 0)

            def patch_row(r, _):
                rvec = jnp.full((NSEG,), r + dy, jnp.int32)
                for t in range(EFF // NSEG):
                    v = plsc.load_gather(staging, [rvec, lanes + (coff + t * NSEG)])
                    pbuf[0, pl.ds(obase + r * EFF + t * NSEG, NSEG)] = v
                return 0

            lax.fori_loop(0, EFF, patch_row, 0)
            if c == 0:
                hs[2] = pltpu.async_copy(
                    images.at[b, 2, pl.ds(y1a, EFF + 8), pl.ds(x1b, 256)],
                    staging.at[pl.ds(0, EFF + 8), pl.ds(0, 256)], sem0)

        if k < NPATCH - 1:
            mx1 = jnp.maximum(x1 - MARGIN, 0)
            my1 = jnp.maximum(y1 - MARGIN, 0)
            mx2 = jnp.minimum(x1 + EFF + MARGIN, W)
            my2 = jnp.minimum(y1 + EFF + MARGIN, H)
            boxes.append((mx1, my1, mx2, my2))
            ibase = jnp.minimum(mx1 // NSEG, NV - 7)
            bbase = jnp.minimum(my1 // BLK, NBLK - 13)

            def mask_blk(tb, _):
                blk = bbase + tb
                full = (blk * BLK >= my1) & ((blk + 1) * BLK <= my2)

                def mask_vreg(ti, _):
                    i = ibase + ti
                    cvec = lanes + i * NSEG
                    colin = (cvec >= mx1) & (cvec < mx2)
                    old = colmax[pl.ds(blk * W + i * NSEG, NSEG)]
                    colmax[pl.ds(blk * W + i * NSEG, NSEG)] = jnp.where(
                        colin & full, NEG, old)
                    return 0

                lax.fori_loop(0, 7, mask_vreg, 0)
                return 0

            lax.fori_loop(0, 13, mask_blk, 0)

            def redo_block(pb, srow):
                pltpu.sync_copy(unc.at[b, pl.ds(pb * BLK, BLK)],
                                staging.at[pl.ds(srow, BLK)])

                def redo_vreg(ti, _):
                    i = ibase + ti
                    cvec = lanes + i * NSEG
                    acc = jnp.full((NSEG,), NEG, jnp.float32)
                    for rr in range(BLK):
                        rvec = jnp.full((NSEG,), srow + rr, jnp.int32)
                        v = plsc.load_gather(staging, [rvec, cvec])
                        rglob = pb * BLK + rr
                        for (bx1, by1, bx2, by2) in boxes:
                            rowin = (rglob >= by1) & (rglob < by2)
                            colin = (cvec >= bx1) & (cvec < bx2)
                            v = jnp.where(colin & rowin, NEG, v)
                        acc = jnp.maximum(acc, v)
                    colmax[pl.ds(pb * W + i * NSEG, NSEG)] = acc
                    return 0

                lax.fori_loop(0, 7, redo_vreg, 0)

            @pl.when(my1 % BLK != 0)
            def _():
                redo_block(my1 // BLK, 0)

            @pl.when(my2 % BLK != 0)
            def _():
                redo_block((my2 - 1) // BLK, BLK)

            def summ_blk(tb, _):
                blk = bbase + tb
                for i in range(NV):
                    v = colmax[pl.ds(blk * W + i * NSEG, NSEG)]
                    sm = v if i == 0 else jnp.maximum(sm, v)
                summ[pl.ds(blk * NSEG, NSEG)] = sm
                return 0

            lax.fori_loop(0, 13, summ_blk, 0)

    cbuf[0, pl.ds(0, NSEG)] = cvacc
    pltpu.sync_copy(pbuf, patches_out.at[b])
    pltpu.sync_copy(cbuf, coords_out.at[b])


def _sc_kernel(images, unc, patches_out, coords_out, staging, colmax, summ,
               pbuf, cbuf, sem0, sem1):
    cid = lax.axis_index("c")
    sid = lax.axis_index("s")
    wid = sid * 2 + cid

    @pl.when(wid < BATCH)
    def _():
        _worker(wid, images, unc, patches_out, coords_out, staging, colmax,
                summ, pbuf, cbuf, sem0, sem1)


@jax.jit
def kernel(images, uncertainty_maps):
    unc = uncertainty_maps.reshape(BATCH, H, W)
    mesh = plsc.VectorSubcoreMesh(
        core_axis_name="c", subcore_axis_name="s", num_cores=2, num_subcores=16)
    patches, coords_raw = pl.kernel(
        _sc_kernel,
        out_type=(
            jax.ShapeDtypeStruct((BATCH, 1, NPATCH * CHANS * PSZ), jnp.float32),
            jax.ShapeDtypeStruct((BATCH, 1, NSEG), jnp.float32),
        ),
        mesh=mesh,
        compiler_params=pltpu.CompilerParams(needs_layout_passes=False),
        scratch_types=[
            pltpu.VMEM((SROWS, W), jnp.float32),
            pltpu.VMEM((NBLK * W,), jnp.float32),
            pltpu.VMEM((NBLK * NSEG,), jnp.float32),
            pltpu.VMEM((1, NPATCH * CHANS * PSZ), jnp.float32),
            pltpu.VMEM((1, NSEG), jnp.float32),
            pltpu.SemaphoreType.DMA,
            pltpu.SemaphoreType.DMA,
        ],
    )(images, unc)
    patches = patches.reshape(BATCH, NPATCH, CHANS, EFF, EFF)
    coords = coords_raw[:, 0, :4 * NPATCH].reshape(BATCH, NPATCH, 4)
    return patches, coords


# speedup vs baseline: 3.1004x; 1.0272x over previous
import functools
import jax
import jax.numpy as jnp
from jax import lax
from jax.experimental import pallas as pl
from jax.experimental.pallas import tpu as pltpu
from jax.experimental.pallas import tpu_sc as plsc

H = 512
W = 512
EFF = 64
HALF = 32
MARGIN = 16
NPATCH = 3
CHANS = 3
BATCH = 16
NSEG = 16
BLK = 8
NBLK = H // BLK
NV = W // NSEG
SROWS = 88
PCHUNK = 128
PSZ = EFF * EFF
NEG = float(-3.4028235e38)
BIGI = 1 << 30


def _worker(b, images, unc, patches_out, coords_out, staging, colmax, summ,
            pbuf, cbuf, sem0, sem1, sem2, sem3, sem4):
    lanes = lax.iota(jnp.int32, NSEG)

    def make_blocks(lblk0, roff):
        def blk_body(lb, _):
            rvecs = [jnp.full((NSEG,), roff + lb * BLK + rr, jnp.int32)
                     for rr in range(BLK)]
            blk = lblk0 + lb
            for i in range(NV):
                cvec = lanes + 16 * i
                acc = jnp.full((NSEG,), NEG, jnp.float32)
                for rr in range(BLK):
                    acc = jnp.maximum(
                        acc, plsc.load_gather(staging, [rvecs[rr], cvec]))
                colmax[pl.ds(blk * W + 16 * i, NSEG)] = acc
                s = acc if i == 0 else jnp.maximum(s, acc)
            summ[pl.ds(blk * NSEG, NSEG)] = s
            return 0

        lax.fori_loop(0, 4, blk_body, 0)

    pltpu.async_copy(unc.at[b, pl.ds(0, 32)], staging.at[pl.ds(0, 32)], sem0)
    pltpu.async_copy(unc.at[b, pl.ds(32, 32)], staging.at[pl.ds(32, 32)], sem1)

    def super_body(j, _):
        pltpu.make_async_copy(unc.at[b, pl.ds(0, 32)],
                              staging.at[pl.ds(0, 32)], sem0).wait()
        make_blocks(8 * j, 0)

        @pl.when(j < 7)
        def _():
            pltpu.async_copy(unc.at[b, pl.ds((2 * j + 2) * 32, 32)],
                             staging.at[pl.ds(0, 32)], sem0)

        pltpu.make_async_copy(unc.at[b, pl.ds(32, 32)],
                              staging.at[pl.ds(32, 32)], sem1).wait()
        make_blocks(8 * j + 4, 32)

        @pl.when(j < 7)
        def _():
            pltpu.async_copy(unc.at[b, pl.ds((2 * j + 3) * 32, 32)],
                             staging.at[pl.ds(32, 32)], sem1)

        return 0

    lax.fori_loop(0, 8, super_body, 0)

    boxes = []
    cvacc = jnp.zeros((NSEG,), jnp.float32)

    for k in range(NPATCH):
        m = jnp.full((NSEG,), NEG, jnp.float32)
        bidx = jnp.zeros((NSEG,), jnp.int32)
        for blk in range(NBLK):
            v = summ[pl.ds(blk * NSEG, NSEG)]
            upd = v > m
            bidx = jnp.where(upd, blk, bidx)
            m = jnp.maximum(m, v)
        g = jnp.max(m)
        bstar = jnp.min(jnp.where(m == g, bidx, BIGI))
        rb = bstar * BLK

        pltpu.sync_copy(unc.at[b, pl.ds(rb, BLK)], staging.at[pl.ds(0, BLK)])

        def find_body(r, carry):
            rbest, cbest = carry
            rvec = jnp.full((NSEG,), r, jnp.int32)
            cmin = jnp.full((NSEG,), BIGI, jnp.int32)
            rglob = rb + r
            for i in range(NV):
                cvec = lanes + 16 * i
                v = plsc.load_gather(staging, [rvec, cvec])
                for (bx1, by1, bx2, by2) in boxes:
                    rowin = (rglob >= by1) & (rglob < by2)
                    colin = (cvec >= bx1) & (cvec < bx2)
                    v = jnp.where(colin & rowin, NEG, v)
                cmin = jnp.minimum(cmin, jnp.where(v == g, cvec, BIGI))
            c_r = jnp.min(cmin)
            hit = (c_r < BIGI) & (rbest == BIGI)
            rbest = jnp.where(hit, rglob, rbest)
            cbest = jnp.where(hit, c_r, cbest)
            return rbest, cbest

        rstar, cstar = lax.fori_loop(
            0, BLK, find_body,
            (jnp.full((), BIGI, jnp.int32), jnp.full((), BIGI, jnp.int32)))

        x1 = jnp.clip(cstar - HALF, 0, W - EFF)
        y1 = jnp.clip(rstar - HALF, 0, H - EFF)

        for off, val in ((0, x1), (1, y1), (2, x1 + EFF), (3, y1 + EFF)):
            cvacc = jnp.where(lanes == (4 * k + off), val.astype(jnp.float32), cvacc)

        if k < NPATCH - 1:
            mx1 = jnp.maximum(x1 - MARGIN, 0)
            my1 = jnp.maximum(y1 - MARGIN, 0)
            mx2 = jnp.minimum(x1 + EFF + MARGIN, W)
            my2 = jnp.minimum(y1 + EFF + MARGIN, H)
            boxes.append((mx1, my1, mx2, my2))
            ibase = jnp.minimum(mx1 // NSEG, NV - 7)
            bbase = jnp.minimum(my1 // BLK, NBLK - 13)
            need1 = (my1 % BLK) != 0
            need2 = (my2 % BLK) != 0
            p1 = my1 // BLK
            p2 = (my2 - 1) // BLK

            @pl.when(need1)
            def _():
                pltpu.async_copy(unc.at[b, pl.ds(p1 * BLK, BLK)],
                                 staging.at[pl.ds(72, BLK)], sem2)

            @pl.when(need2)
            def _():
                pltpu.async_copy(unc.at[b, pl.ds(p2 * BLK, BLK)],
                                 staging.at[pl.ds(80, BLK)], sem3)

        y1a = (y1 // 8) * 8
        dy = y1 - y1a
        x1b = jnp.minimum((x1 // 128) * 128, W - 256)
        dx = x1 - x1b
        hs = [
            pltpu.async_copy(images.at[b, 0, pl.ds(y1a, EFF + 8), pl.ds(x1b, 256)],
                             staging.at[pl.ds(0, EFF + 8), pl.ds(0, 256)], sem0),
            pltpu.async_copy(images.at[b, 1, pl.ds(y1a, EFF + 8), pl.ds(x1b, 256)],
                             staging.at[pl.ds(0, EFF + 8), pl.ds(256, 256)], sem1),
            None,
        ]
        for c in range(CHANS):
            hs[c].wait()
            obase = (k * CHANS + c) * PSZ
            coff = dx + (256 if c == 1 else 0)

            def patch_row(r, _):
                rvec = jnp.full((NSEG,), r + dy, jnp.int32)
                for t in range(EFF // NSEG):
                    v = plsc.load_gather(staging, [rvec, lanes + (coff + t * NSEG)])
                    pbuf[0, pl.ds(obase + r * EFF + t * NSEG, NSEG)] = v
                return 0

            lax.fori_loop(0, EFF, patch_row, 0)
            if c == CHANS - 1:
                pltpu.async_copy(
                    pbuf.at[0, pl.ds(k * CHANS * PSZ, CHANS * PSZ)],
                    patches_out.at[b, 0, pl.ds(k * CHANS * PSZ, CHANS * PSZ)],
                    sem4)
            if c == 0:
                hs[2] = pltpu.async_copy(
                    images.at[b, 2, pl.ds(y1a, EFF + 8), pl.ds(x1b, 256)],
                    staging.at[pl.ds(0, EFF + 8), pl.ds(0, 256)], sem0)

        if k < NPATCH - 1:

            def mask_blk(tb, _):
                blk = bbase + tb
                full = (blk * BLK >= my1) & ((blk + 1) * BLK <= my2)

                def mask_vreg(ti, _):
                    i = ibase + ti
                    cvec = lanes + i * NSEG
                    colin = (cvec >= mx1) & (cvec < mx2)
                    old = colmax[pl.ds(blk * W + i * NSEG, NSEG)]
                    colmax[pl.ds(blk * W + i * NSEG, NSEG)] = jnp.where(
                        colin & full, NEG, old)
                    return 0

                lax.fori_loop(0, 7, mask_vreg, 0)
                return 0

            lax.fori_loop(0, 13, mask_blk, 0)

            def redo_block(pb, srow, sem):
                pltpu.make_async_copy(unc.at[b, pl.ds(pb * BLK, BLK)],
                                      staging.at[pl.ds(srow, BLK)], sem).wait()

                def redo_vreg(ti, _):
                    i = ibase + ti
                    cvec = lanes + i * NSEG
                    acc = jnp.full((NSEG,), NEG, jnp.float32)
                    for rr in range(BLK):
                        rvec = jnp.full((NSEG,), srow + rr, jnp.int32)
                        v = plsc.load_gather(staging, [rvec, cvec])
                        rglob = pb * BLK + rr
                        for (bx1, by1, bx2, by2) in boxes:
                            rowin = (rglob >= by1) & (rglob < by2)
                            colin = (cvec >= bx1) & (cvec < bx2)
                            v = jnp.where(colin & rowin, NEG, v)
                        acc = jnp.maximum(acc, v)
                    colmax[pl.ds(pb * W + i * NSEG, NSEG)] = acc
                    return 0

                lax.fori_loop(0, 7, redo_vreg, 0)

            @pl.when(need1)
            def _():
                redo_block(p1, 72, sem2)

            @pl.when(need2)
            def _():
                redo_block(p2, 80, sem3)

            def summ_blk(tb, _):
                blk = bbase + tb
                for i in range(NV):
                    v = colmax[pl.ds(blk * W + i * NSEG, NSEG)]
                    sm = v if i == 0 else jnp.maximum(sm, v)
                summ[pl.ds(blk * NSEG, NSEG)] = sm
                return 0

            lax.fori_loop(0, 13, summ_blk, 0)

    cbuf[0, pl.ds(0, NSEG)] = cvacc
    for k in range(NPATCH):
        pltpu.make_async_copy(pbuf.at[0, pl.ds(k * CHANS * PSZ, CHANS * PSZ)],
                              patches_out.at[b, 0, pl.ds(k * CHANS * PSZ,
                                                         CHANS * PSZ)],
                              sem4).wait()
    pltpu.sync_copy(cbuf, coords_out.at[b])


def _sc_kernel(images, unc, patches_out, coords_out, staging, colmax, summ,
               pbuf, cbuf, sem0, sem1, sem2, sem3, sem4):
    cid = lax.axis_index("c")
    sid = lax.axis_index("s")
    wid = sid * 2 + cid

    @pl.when(wid < BATCH)
    def _():
        _worker(wid, images, unc, patches_out, coords_out, staging, colmax,
                summ, pbuf, cbuf, sem0, sem1, sem2, sem3, sem4)


@jax.jit
def kernel(images, uncertainty_maps):
    unc = uncertainty_maps.reshape(BATCH, H, W)
    mesh = plsc.VectorSubcoreMesh(
        core_axis_name="c", subcore_axis_name="s", num_cores=2, num_subcores=16)
    patches, coords_raw = pl.kernel(
        _sc_kernel,
        out_type=(
            jax.ShapeDtypeStruct((BATCH, 1, NPATCH * CHANS * PSZ), jnp.float32),
            jax.ShapeDtypeStruct((BATCH, 1, NSEG), jnp.float32),
        ),
        mesh=mesh,
        compiler_params=pltpu.CompilerParams(needs_layout_passes=False),
        scratch_types=[
            pltpu.VMEM((SROWS, W), jnp.float32),
            pltpu.VMEM((NBLK * W,), jnp.float32),
            pltpu.VMEM((NBLK * NSEG,), jnp.float32),
            pltpu.VMEM((1, NPATCH * CHANS * PSZ), jnp.float32),
            pltpu.VMEM((1, NSEG), jnp.float32),
            pltpu.SemaphoreType.DMA,
            pltpu.SemaphoreType.DMA,
            pltpu.SemaphoreType.DMA,
            pltpu.SemaphoreType.DMA,
            pltpu.SemaphoreType.DMA,
        ],
    )(images, unc)
    patches = patches.reshape(BATCH, NPATCH, CHANS, EFF, EFF)
    coords = coords_raw[:, 0, :4 * NPATCH].reshape(BATCH, NPATCH, 4)
    return patches, coords


# speedup vs baseline: 3.1034x; 1.0010x over previous
import functools
import jax
import jax.numpy as jnp
from jax import lax
from jax.experimental import pallas as pl
from jax.experimental.pallas import tpu as pltpu
from jax.experimental.pallas import tpu_sc as plsc

H = 512
W = 512
EFF = 64
HALF = 32
MARGIN = 16
NPATCH = 3
CHANS = 3
BATCH = 16
NSEG = 16
BLK = 8
NBLK = H // BLK
NV = W // NSEG
SROWS = 88
PCHUNK = 128
PSZ = EFF * EFF
NEG = float(-3.4028235e38)
BIGI = 1 << 30


def _worker(b, images, unc, patches_out, coords_out, staging, colmax, colrow,
            summ, pbuf, cbuf, sem0, sem1, sem2, sem3, sem4):
    lanes = lax.iota(jnp.int32, NSEG)

    def make_blocks(lblk0, roff):
        def blk_body(lb, _):
            rvecs = [jnp.full((NSEG,), roff + lb * BLK + rr, jnp.int32)
                     for rr in range(BLK)]
            blk = lblk0 + lb
            for i in range(NV):
                cvec = lanes + 16 * i
                acc = jnp.full((NSEG,), NEG, jnp.float32)
                racc = jnp.zeros((NSEG,), jnp.int32)
                for rr in range(BLK):
                    v = plsc.load_gather(staging, [rvecs[rr], cvec])
                    upd = v > acc
                    racc = jnp.where(upd, blk * BLK + rr, racc)
                    acc = jnp.maximum(acc, v)
                colmax[pl.ds(blk * W + 16 * i, NSEG)] = acc
                colrow[pl.ds(blk * W + 16 * i, NSEG)] = racc
                s = acc if i == 0 else jnp.maximum(s, acc)
            summ[pl.ds(blk * NSEG, NSEG)] = s
            return 0

        lax.fori_loop(0, 4, blk_body, 0)

    pltpu.async_copy(unc.at[b, pl.ds(0, 32)], staging.at[pl.ds(0, 32)], sem0)
    pltpu.async_copy(unc.at[b, pl.ds(32, 32)], staging.at[pl.ds(32, 32)], sem1)

    def super_body(j, _):
        pltpu.make_async_copy(unc.at[b, pl.ds(0, 32)],
                              staging.at[pl.ds(0, 32)], sem0).wait()
        make_blocks(8 * j, 0)

        @pl.when(j < 7)
        def _():
            pltpu.async_copy(unc.at[b, pl.ds((2 * j + 2) * 32, 32)],
                             staging.at[pl.ds(0, 32)], sem0)

        pltpu.make_async_copy(unc.at[b, pl.ds(32, 32)],
                              staging.at[pl.ds(32, 32)], sem1).wait()
        make_blocks(8 * j + 4, 32)

        @pl.when(j < 7)
        def _():
            pltpu.async_copy(unc.at[b, pl.ds((2 * j + 3) * 32, 32)],
                             staging.at[pl.ds(32, 32)], sem1)

        return 0

    lax.fori_loop(0, 8, super_body, 0)

    boxes = []
    cvacc = jnp.zeros((NSEG,), jnp.float32)

    for k in range(NPATCH):
        m = jnp.full((NSEG,), NEG, jnp.float32)
        bidx = jnp.zeros((NSEG,), jnp.int32)
        for blk in range(NBLK):
            v = summ[pl.ds(blk * NSEG, NSEG)]
            upd = v > m
            bidx = jnp.where(upd, blk, bidx)
            m = jnp.maximum(m, v)
        g = jnp.max(m)
        bstar = jnp.min(jnp.where(m == g, bidx, BIGI))
        rb = bstar * BLK

        rmin = jnp.full((NSEG,), BIGI, jnp.int32)
        for i in range(NV):
            cm = colmax[pl.ds(bstar * W + 16 * i, NSEG)]
            rv = colrow[pl.ds(bstar * W + 16 * i, NSEG)]
            rmin = jnp.minimum(rmin, jnp.where(cm == g, rv, BIGI))
        rstar = jnp.min(rmin)
        cmin = jnp.full((NSEG,), BIGI, jnp.int32)
        for i in range(NV):
            cm = colmax[pl.ds(bstar * W + 16 * i, NSEG)]
            rv = colrow[pl.ds(bstar * W + 16 * i, NSEG)]
            cvec = lanes + 16 * i
            cmin = jnp.minimum(
                cmin, jnp.where((cm == g) & (rv == rstar), cvec, BIGI))
        cstar = jnp.min(cmin)

        x1 = jnp.clip(cstar - HALF, 0, W - EFF)
        y1 = jnp.clip(rstar - HALF, 0, H - EFF)

        for off, val in ((0, x1), (1, y1), (2, x1 + EFF), (3, y1 + EFF)):
            cvacc = jnp.where(lanes == (4 * k + off), val.astype(jnp.float32), cvacc)

        if k < NPATCH - 1:
            mx1 = jnp.maximum(x1 - MARGIN, 0)
            my1 = jnp.maximum(y1 - MARGIN, 0)
            mx2 = jnp.minimum(x1 + EFF + MARGIN, W)
            my2 = jnp.minimum(y1 + EFF + MARGIN, H)
            boxes.append((mx1, my1, mx2, my2))
            ibase = jnp.minimum(mx1 // NSEG, NV - 7)
            bbase = jnp.minimum(my1 // BLK, NBLK - 13)
            need1 = (my1 % BLK) != 0
            need2 = (my2 % BLK) != 0
            p1 = my1 // BLK
            p2 = (my2 - 1) // BLK

            @pl.when(need1)
            def _():
                pltpu.async_copy(unc.at[b, pl.ds(p1 * BLK, BLK)],
                                 staging.at[pl.ds(72, BLK)], sem2)

            @pl.when(need2)
            def _():
                pltpu.async_copy(unc.at[b, pl.ds(p2 * BLK, BLK)],
                                 staging.at[pl.ds(80, BLK)], sem3)

        y1a = (y1 // 8) * 8
        dy = y1 - y1a
        x1b = jnp.minimum((x1 // 128) * 128, W - 256)
        dx = x1 - x1b
        hs = [
            pltpu.async_copy(images.at[b, 0, pl.ds(y1a, EFF + 8), pl.ds(x1b, 256)],
                             staging.at[pl.ds(0, EFF + 8), pl.ds(0, 256)], sem0),
            pltpu.async_copy(images.at[b, 1, pl.ds(y1a, EFF + 8), pl.ds(x1b, 256)],
                             staging.at[pl.ds(0, EFF + 8), pl.ds(256, 256)], sem1),
            None,
        ]
        if k > 0:
            pltpu.make_async_copy(
                pbuf.at[0],
                patches_out.at[b, 0, pl.ds((k - 1) * CHANS * PSZ, CHANS * PSZ)],
                sem4).wait()
        for c in range(CHANS):
            hs[c].wait()
            obase = c * PSZ
            coff = dx + (256 if c == 1 else 0)

            def patch_row(r, _):
                rvec = jnp.full((NSEG,), r + dy, jnp.int32)
                for t in range(EFF // NSEG):
                    v = plsc.load_gather(staging, [rvec, lanes + (coff + t * NSEG)])
                    pbuf[0, pl.ds(obase + r * EFF + t * NSEG, NSEG)] = v
                return 0

            lax.fori_loop(0, EFF, patch_row, 0)
            if c == CHANS - 1:
                pltpu.async_copy(
                    pbuf.at[0],
                    patches_out.at[b, 0, pl.ds(k * CHANS * PSZ, CHANS * PSZ)],
                    sem4)
            if c == 0:
                hs[2] = pltpu.async_copy(
                    images.at[b, 2, pl.ds(y1a, EFF + 8), pl.ds(x1b, 256)],
                    staging.at[pl.ds(0, EFF + 8), pl.ds(0, 256)], sem0)

        if k < NPATCH - 1:

            def mask_blk(tb, _):
                blk = bbase + tb
                full = (blk * BLK >= my1) & ((blk + 1) * BLK <= my2)

                def mask_vreg(ti, _):
                    i = ibase + ti
                    cvec = lanes + i * NSEG
                    colin = (cvec >= mx1) & (cvec < mx2)
                    old = colmax[pl.ds(blk * W + i * NSEG, NSEG)]
                    colmax[pl.ds(blk * W + i * NSEG, NSEG)] = jnp.where(
                        colin & full, NEG, old)
                    return 0

                lax.fori_loop(0, 7, mask_vreg, 0)
                return 0

            lax.fori_loop(0, 13, mask_blk, 0)

            def redo_block(pb, srow, sem):
                pltpu.make_async_copy(unc.at[b, pl.ds(pb * BLK, BLK)],
                                      staging.at[pl.ds(srow, BLK)], sem).wait()

                def redo_vreg(ti, _):
                    i = ibase + ti
                    cvec = lanes + i * NSEG
                    acc = jnp.full((NSEG,), NEG, jnp.float32)
                    racc = jnp.zeros((NSEG,), jnp.int32)
                    for rr in range(BLK):
                        rvec = jnp.full((NSEG,), srow + rr, jnp.int32)
                        v = plsc.load_gather(staging, [rvec, cvec])
                        rglob = pb * BLK + rr
                        for (bx1, by1, bx2, by2) in boxes:
                            rowin = (rglob >= by1) & (rglob < by2)
                            colin = (cvec >= bx1) & (cvec < bx2)
                            v = jnp.where(colin & rowin, NEG, v)
                        upd = v > acc
                        racc = jnp.where(upd, rglob, racc)
                        acc = jnp.maximum(acc, v)
                    colmax[pl.ds(pb * W + i * NSEG, NSEG)] = acc
                    colrow[pl.ds(pb * W + i * NSEG, NSEG)] = racc
                    return 0

                lax.fori_loop(0, 7, redo_vreg, 0)

            @pl.when(need1)
            def _():
                redo_block(p1, 72, sem2)

            @pl.when(need2)
            def _():
                redo_block(p2, 80, sem3)

            def summ_blk(tb, _):
                blk = bbase + tb
                for i in range(NV):
                    v = colmax[pl.ds(blk * W + i * NSEG, NSEG)]
                    sm = v if i == 0 else jnp.maximum(sm, v)
                summ[pl.ds(blk * NSEG, NSEG)] = sm
                return 0

            lax.fori_loop(0, 13, summ_blk, 0)

    cbuf[0, pl.ds(0, NSEG)] = cvacc
    pltpu.make_async_copy(
        pbuf.at[0],
        patches_out.at[b, 0, pl.ds((NPATCH - 1) * CHANS * PSZ, CHANS * PSZ)],
        sem4).wait()
    pltpu.sync_copy(cbuf, coords_out.at[b])


def _sc_kernel(images, unc, patches_out, coords_out, staging, colmax, colrow,
               summ, pbuf, cbuf, sem0, sem1, sem2, sem3, sem4):
    cid = lax.axis_index("c")
    sid = lax.axis_index("s")
    wid = sid * 2 + cid

    @pl.when(wid < BATCH)
    def _():
        _worker(wid, images, unc, patches_out, coords_out, staging, colmax,
                colrow, summ, pbuf, cbuf, sem0, sem1, sem2, sem3, sem4)


@jax.jit
def kernel(images, uncertainty_maps):
    unc = uncertainty_maps.reshape(BATCH, H, W)
    mesh = plsc.VectorSubcoreMesh(
        core_axis_name="c", subcore_axis_name="s", num_cores=2, num_subcores=16)
    patches, coords_raw = pl.kernel(
        _sc_kernel,
        out_type=(
            jax.ShapeDtypeStruct((BATCH, 1, NPATCH * CHANS * PSZ), jnp.float32),
            jax.ShapeDtypeStruct((BATCH, 1, NSEG), jnp.float32),
        ),
        mesh=mesh,
        compiler_params=pltpu.CompilerParams(needs_layout_passes=False),
        scratch_types=[
            pltpu.VMEM((SROWS, W), jnp.float32),
            pltpu.VMEM((NBLK * W,), jnp.float32),
            pltpu.VMEM((NBLK * W,), jnp.int32),
            pltpu.VMEM((NBLK * NSEG,), jnp.float32),
            pltpu.VMEM((1, CHANS * PSZ), jnp.float32),
            pltpu.VMEM((1, NSEG), jnp.float32),
            pltpu.SemaphoreType.DMA,
            pltpu.SemaphoreType.DMA,
            pltpu.SemaphoreType.DMA,
            pltpu.SemaphoreType.DMA,
            pltpu.SemaphoreType.DMA,
        ],
    )(images, unc)
    patches = patches.reshape(BATCH, NPATCH, CHANS, EFF, EFF)
    coords = coords_raw[:, 0, :4 * NPATCH].reshape(BATCH, NPATCH, 4)
    return patches, coords
